# Initial kernel scaffold; baseline (speedup 1.0000x reference)
import jax
import jax.numpy as jnp
from jax.experimental import pallas as pl


def kernel(x, table):
    raise NotImplementedError("write your pallas kernel here")


# speedup vs baseline: 1.1027x; 1.1027x over previous
import functools

import jax
import jax.numpy as jnp
from jax import lax
from jax.experimental import pallas as pl
from jax.experimental.pallas import tpu as pltpu
from jax.experimental.pallas import tpu_sc as plsc

_NC = 2
_NS = 16
_NW = _NC * _NS


def _gather_rows(table, idx, chunk):
    (n,) = idx.shape
    _, d = table.shape
    b_per_w = n // _NW
    n_chunks = b_per_w // chunk
    mesh = plsc.VectorSubcoreMesh(core_axis_name="c", subcore_axis_name="s")

    @functools.partial(
        pl.kernel,
        out_type=jax.ShapeDtypeStruct((n, d), table.dtype),
        mesh=mesh,
        scratch_types=[
            pltpu.VMEM((chunk,), jnp.int32),
            pltpu.VMEM((chunk, d), jnp.float32),
            pltpu.SemaphoreType.DMA,
        ],
        compiler_params=pltpu.CompilerParams(use_tc_tiling_on_sc=False),
    )
    def k(table_hbm, idx_hbm, out_hbm, idx_v, rows_v, sem):
        wid = lax.axis_index("s") * _NC + lax.axis_index("c")
        base = wid * b_per_w

        def body(j, carry):
            off = base + j * chunk
            pltpu.sync_copy(idx_hbm.at[pl.ds(off, chunk)], idx_v)
            pltpu.async_copy(table_hbm.at[idx_v], rows_v, sem).wait()
            pltpu.sync_copy(rows_v, out_hbm.at[pl.ds(off, chunk)])
            return carry

        lax.fori_loop(0, n_chunks, body, 0)

    return k(table, idx)


def kernel(x, table):
    b, h = x.shape
    _, d = table.shape
    idx = x.reshape(b * h).astype(jnp.int32)
    out = _gather_rows(table, idx, chunk=1600)
    return out.reshape(b, h, d)


# speedup vs baseline: 1.1109x; 1.0074x over previous
import functools

import jax
import jax.numpy as jnp
from jax import lax
from jax.experimental import pallas as pl
from jax.experimental.pallas import tpu as pltpu
from jax.experimental.pallas import tpu_sc as plsc

_NC = 2
_NS = 16
_NW = _NC * _NS


def _gather_rows(table, idx, chunk):
    (n,) = idx.shape
    _, d = table.shape
    b_per_w = n // _NW
    n_chunks = b_per_w // chunk
    mesh = plsc.VectorSubcoreMesh(core_axis_name="c", subcore_axis_name="s")

    @functools.partial(
        pl.kernel,
        out_type=jax.ShapeDtypeStruct((n, d), table.dtype),
        mesh=mesh,
        scratch_types=[
            pltpu.VMEM((2, chunk), jnp.int32),
            pltpu.VMEM((2, chunk, d), jnp.float32),
            pltpu.SemaphoreType.DMA,
            pltpu.SemaphoreType.DMA,
            pltpu.SemaphoreType.DMA,
        ],
        compiler_params=pltpu.CompilerParams(use_tc_tiling_on_sc=False),
    )
    def k(table_hbm, idx_hbm, out_hbm, idx_v, rows_v, isem, gsem, osem):
        wid = lax.axis_index("s") * _NC + lax.axis_index("c")
        base = wid * b_per_w

        def idx_copy(j, slot):
            return pltpu.make_async_copy(
                idx_hbm.at[pl.ds(base + j * chunk, chunk)], idx_v.at[slot], isem)

        def gather_copy(slot):
            return pltpu.make_async_copy(
                table_hbm.at[idx_v.at[slot]], rows_v.at[slot], gsem)

        def out_copy(j, slot):
            return pltpu.make_async_copy(
                rows_v.at[slot], out_hbm.at[pl.ds(base + j * chunk, chunk)], osem)

        idx_copy(0, 0).start()
        idx_copy(0, 0).wait()
        gather_copy(0).start()
        idx_copy(1, 1).start()

        def body(j, carry):
            cur = lax.rem(j, 2)
            nxt = 1 - cur
            gather_copy(cur).wait()

            @pl.when(j >= 1)
            def _():
                out_copy(j - 1, nxt).wait()

            @pl.when(j < n_chunks - 1)
            def _():
                idx_copy(j + 1, nxt).wait()
                gather_copy(nxt).start()

            out_copy(j, cur).start()

            @pl.when(j < n_chunks - 2)
            def _():
                idx_copy(j + 2, cur).start()

            return carry

        lax.fori_loop(0, n_chunks, body, 0)
        out_copy(n_chunks - 1, lax.rem(n_chunks - 1, 2)).wait()

    return k(table, idx)


def kernel(x, table):
    b, h = x.shape
    _, d = table.shape
    idx = x.reshape(b * h).astype(jnp.int32)
    out = _gather_rows(table, idx, chunk=1600)
    return out.reshape(b, h, d)
